# baseline (speedup 1.0000x reference)
import jax
import jax.numpy as jnp
from jax.experimental import pallas as pl
from jax.experimental.pallas import tpu as pltpu

_THRESHOLD = 0.5
_FOCAL_ALPHA = 0.25
_FOCAL_GAMMA = 2.0
_REG_WEIGHT = 1.0
_NCHUNK = 3


def _fused_kernel(boxes_ref, labels_ref, pcx_ref, pcy_ref, pw_ref, ph_ref,
                  l0_ref, l1_ref, l2_ref, l3_ref, scores_ref,
                  npos_ref, locsum_ref, fl_ref, m_ref, cls_scr):
    j = pl.program_id(0)

    @pl.when(j == 0)
    def _():
        npos_ref[...] = jnp.zeros((1, 1), jnp.float32)
        locsum_ref[...] = jnp.zeros((1, 1), jnp.float32)
        fl_ref[...] = jnp.zeros((1, 1), jnp.float32)
        m_ref[...] = jnp.zeros((1, 1), jnp.float32)

    @pl.when(j % 2 == 0)
    def _match():
        nobj = boxes_ref.shape[1]
        pcx = pcx_ref[...]
        pcy = pcy_ref[...]
        pw = pw_ref[...]
        ph = ph_ref[...]
        px1 = pcx - pw * 0.5
        py1 = pcy - ph * 0.5
        px2 = pcx + pw * 0.5
        py2 = pcy + ph * 0.5

        b = boxes_ref[0]
        bx1 = b[:, 0:1]
        by1 = b[:, 1:2]
        bx2 = b[:, 2:3]
        by2 = b[:, 3:4]

        wx = jnp.maximum(jnp.minimum(bx2, px2) - jnp.maximum(bx1, px1), 0.0)
        wy = jnp.maximum(jnp.minimum(by2, py2) - jnp.maximum(by1, py1), 0.0)
        inter = wx * wy
        a1 = (bx2 - bx1) * (by2 - by1)
        ov = inter / (a1 + pw * ph - inter)

        npriors = ov.shape[1]
        iota_o = jax.lax.broadcasted_iota(jnp.int32, ov.shape, 0)
        iota_p = jax.lax.broadcasted_iota(jnp.int32, ov.shape, 1)

        maxv = jnp.max(ov, axis=0, keepdims=True)
        obj_idx = jnp.min(jnp.where(ov == maxv, iota_o, nobj),
                          axis=0, keepdims=True)
        rowmax = jnp.max(ov, axis=1, keepdims=True)
        pfo = jnp.min(jnp.where(ov == rowmax, iota_p, npriors),
                      axis=1, keepdims=True)

        match = iota_p == pfo
        chosen = jnp.max(jnp.where(match, iota_o, -1), axis=0, keepdims=True)
        forced = chosen >= 0
        obj_final = jnp.where(forced, chosen, obj_idx)
        ovl_final = jnp.where(forced, 1.0, maxv)

        sel = iota_o == obj_final
        lab = labels_ref[0]
        lab_g = jnp.sum(jnp.where(sel, lab, 0), axis=0, keepdims=True)

        pos = ovl_final >= _THRESHOLD
        neg = ovl_final < _THRESHOLD - 0.1
        cls_scr[...] = jnp.where(pos, lab_g, jnp.where(neg, 0, -1))

        gx1 = jnp.sum(jnp.where(sel, bx1, 0.0), axis=0, keepdims=True)
        gy1 = jnp.sum(jnp.where(sel, by1, 0.0), axis=0, keepdims=True)
        gx2 = jnp.sum(jnp.where(sel, bx2, 0.0), axis=0, keepdims=True)
        gy2 = jnp.sum(jnp.where(sel, by2, 0.0), axis=0, keepdims=True)
        bcx = (gx1 + gx2) * 0.5
        bcy = (gy1 + gy2) * 0.5
        t0 = (bcx - pcx) / pw * 10.0
        t1 = (bcy - pcy) / ph * 10.0
        t2 = jnp.log((gx2 - gx1) / pw) * 5.0
        t3 = jnp.log((gy2 - gy1) / ph) * 5.0

        posf = pos.astype(jnp.float32)
        ld = (jnp.abs(l0_ref[0] - t0) + jnp.abs(l1_ref[0] - t1)
              + jnp.abs(l2_ref[0] - t2) + jnp.abs(l3_ref[0] - t3)) * posf

        npos_ref[...] += jnp.sum(posf).reshape(1, 1)
        locsum_ref[...] += jnp.sum(ld).reshape(1, 1)

    @pl.when(j % 2 == 1)
    def _focal():
        _, nrow, grp, ncls = scores_ref.shape
        npriors = nrow * grp
        chunk = npriors // _NCHUNK
        s2 = scores_ref[...].reshape(npriors, ncls)
        fl_tot = jnp.zeros((), jnp.float32)
        m_tot = jnp.zeros((), jnp.float32)
        for k in range(_NCHUNK):
            st = jnp.transpose(s2[chunk * k:chunk * (k + 1), :])
            cls = cls_scr[:, chunk * k:chunk * (k + 1)]
            mx = jnp.max(st, axis=0, keepdims=True)
            e = jnp.exp(st - mx)
            se = jnp.sum(e, axis=0, keepdims=True)
            lse = mx + jnp.log(se)
            iota_c = jax.lax.broadcasted_iota(jnp.int32, st.shape, 0)
            sv = jnp.sum(jnp.where(iota_c == jnp.maximum(cls, 0), st, 0.0),
                         axis=0, keepdims=True)
            lpt = sv - lse
            pt = jnp.exp(lpt)
            mm = (cls >= 0).astype(jnp.float32)
            alpha = jnp.where(cls > 0, _FOCAL_ALPHA, 1.0 - _FOCAL_ALPHA)
            om = 1.0 - pt
            fl_tot += jnp.sum(-alpha * om * om * lpt * mm)
            m_tot += jnp.sum(mm)
        fl_ref[...] += fl_tot.reshape(1, 1)
        m_ref[...] += m_tot.reshape(1, 1)


def kernel(predicted_locs, predicted_scores, boxes, priors_cxcy, labels):
    B, P, C = predicted_scores.shape
    NOBJ = boxes.shape[1]
    GRP = 8
    s4 = predicted_scores.reshape(B, P // GRP, GRP, C)

    pr = [priors_cxcy[:, k].reshape(1, P) for k in range(4)]
    lc = [predicted_locs[:, :, k].reshape(B, 1, P) for k in range(4)]
    labels3 = labels.astype(jnp.int32)[..., None]

    npos, locsum, fl_sum, m_sum = pl.pallas_call(
        _fused_kernel,
        grid=(2 * B,),
        in_specs=[
            pl.BlockSpec((1, NOBJ, 4), lambda j: (j // 2, 0, 0)),
            pl.BlockSpec((1, NOBJ, 1), lambda j: (j // 2, 0, 0)),
        ] + [pl.BlockSpec((1, P), lambda j: (0, 0)) for _ in range(4)]
          + [pl.BlockSpec((1, 1, P), lambda j: (j // 2, 0, 0)) for _ in range(4)]
          + [pl.BlockSpec((1, P // GRP, GRP, C), lambda j: (j // 2, 0, 0, 0))],
        out_specs=[pl.BlockSpec((1, 1), lambda j: (0, 0)) for _ in range(4)],
        out_shape=[jax.ShapeDtypeStruct((1, 1), jnp.float32) for _ in range(4)],
        scratch_shapes=[pltpu.VMEM((1, P), jnp.int32)],
    )(boxes, labels3, *pr, *lc, s4)

    conf_loss = fl_sum[0, 0] / jnp.maximum(m_sum[0, 0], 1.0)
    loc_loss = locsum[0, 0] / jnp.maximum(npos[0, 0] * 4.0, 1.0)
    return conf_loss + _REG_WEIGHT * loc_loss


# speedup vs baseline: 1.2446x; 1.2446x over previous
import jax
import jax.numpy as jnp
from jax.experimental import pallas as pl

_THRESHOLD = 0.5
_FOCAL_ALPHA = 0.25
_FOCAL_GAMMA = 2.0
_REG_WEIGHT = 1.0


def _match_kernel(boxes_ref, labels_ref, pcx_ref, pcy_ref, pw_ref, ph_ref,
                  l0_ref, l1_ref, l2_ref, l3_ref,
                  cls_ref, npos_ref, locsum_ref):
    i = pl.program_id(0)
    nobj = boxes_ref.shape[1]

    pcx = pcx_ref[...]
    pcy = pcy_ref[...]
    pw = pw_ref[...]
    ph = ph_ref[...]
    px1 = pcx - pw * 0.5
    py1 = pcy - ph * 0.5
    px2 = pcx + pw * 0.5
    py2 = pcy + ph * 0.5

    b = boxes_ref[0]
    bx1 = b[:, 0:1]
    by1 = b[:, 1:2]
    bx2 = b[:, 2:3]
    by2 = b[:, 3:4]

    wx = jnp.maximum(jnp.minimum(bx2, px2) - jnp.maximum(bx1, px1), 0.0)
    wy = jnp.maximum(jnp.minimum(by2, py2) - jnp.maximum(by1, py1), 0.0)
    inter = wx * wy
    a1 = (bx2 - bx1) * (by2 - by1)
    a2 = pw * ph
    ov = inter / (a1 + a2 - inter)

    npriors = ov.shape[1]
    iota_o = jax.lax.broadcasted_iota(jnp.int32, ov.shape, 0)
    iota_p = jax.lax.broadcasted_iota(jnp.int32, ov.shape, 1)

    maxv = jnp.max(ov, axis=0, keepdims=True)
    obj_idx = jnp.min(jnp.where(ov == maxv, iota_o, nobj),
                      axis=0, keepdims=True)
    rowmax = jnp.max(ov, axis=1, keepdims=True)
    pfo = jnp.min(jnp.where(ov == rowmax, iota_p, npriors),
                  axis=1, keepdims=True)

    match = iota_p == pfo
    chosen = jnp.max(jnp.where(match, iota_o, -1), axis=0, keepdims=True)
    forced = chosen >= 0
    obj_final = jnp.where(forced, chosen, obj_idx)
    ovl_final = jnp.where(forced, 1.0, maxv)

    sel = iota_o == obj_final
    lab = labels_ref[0]
    lab_g = jnp.sum(jnp.where(sel, lab, 0), axis=0, keepdims=True)

    pos = ovl_final >= _THRESHOLD
    neg = ovl_final < _THRESHOLD - 0.1
    cls_m = jnp.where(pos, lab_g, jnp.where(neg, 0, -1))
    cls_ref[0, :, :] = cls_m

    gx1 = jnp.sum(jnp.where(sel, bx1, 0.0), axis=0, keepdims=True)
    gy1 = jnp.sum(jnp.where(sel, by1, 0.0), axis=0, keepdims=True)
    gx2 = jnp.sum(jnp.where(sel, bx2, 0.0), axis=0, keepdims=True)
    gy2 = jnp.sum(jnp.where(sel, by2, 0.0), axis=0, keepdims=True)
    bcx = (gx1 + gx2) * 0.5
    bcy = (gy1 + gy2) * 0.5
    bw = gx2 - gx1
    bh = gy2 - gy1
    t0 = (bcx - pcx) / pw * 10.0
    t1 = (bcy - pcy) / ph * 10.0
    t2 = jnp.log(bw / pw) * 5.0
    t3 = jnp.log(bh / ph) * 5.0

    l0 = l0_ref[0]
    l1 = l1_ref[0]
    l2 = l2_ref[0]
    l3 = l3_ref[0]
    posf = pos.astype(jnp.float32)
    ld = (jnp.abs(l0 - t0) + jnp.abs(l1 - t1)
          + jnp.abs(l2 - t2) + jnp.abs(l3 - t3)) * posf

    @pl.when(i == 0)
    def _():
        npos_ref[...] = jnp.zeros((1, 1), jnp.float32)
        locsum_ref[...] = jnp.zeros((1, 1), jnp.float32)

    npos_ref[...] += jnp.sum(posf).reshape(1, 1)
    locsum_ref[...] += jnp.sum(ld).reshape(1, 1)


def _focal_kernel(s0_ref, s1_ref, s2_ref, c0_ref, c1_ref, c2_ref,
                  fl_ref, m_ref):
    j = pl.program_id(0)
    fl_tot = jnp.zeros((), jnp.float32)
    m_tot = jnp.zeros((), jnp.float32)
    for s_ref, c_ref in ((s0_ref, c0_ref), (s1_ref, c1_ref), (s2_ref, c2_ref)):
        tpr, grp, ncls = s_ref.shape
        s = s_ref[...].reshape(tpr * grp, ncls)
        st = jnp.transpose(s)
        cls = c_ref[0]
        mx = jnp.max(st, axis=0, keepdims=True)
        e = jnp.exp(st - mx)
        se = jnp.sum(e, axis=0, keepdims=True)
        lse = mx + jnp.log(se)
        iota_c = jax.lax.broadcasted_iota(jnp.int32, st.shape, 0)
        sv = jnp.sum(jnp.where(iota_c == jnp.maximum(cls, 0), st, 0.0),
                     axis=0, keepdims=True)
        lpt = sv - lse
        pt = jnp.exp(lpt)
        mm = (cls >= 0).astype(jnp.float32)
        alpha = jnp.where(cls > 0, _FOCAL_ALPHA, 1.0 - _FOCAL_ALPHA)
        om = 1.0 - pt
        fl_tot += jnp.sum(-alpha * om * om * lpt * mm)
        m_tot += jnp.sum(mm)

    @pl.when(j == 0)
    def _():
        fl_ref[...] = jnp.zeros((1, 1), jnp.float32)
        m_ref[...] = jnp.zeros((1, 1), jnp.float32)

    fl_ref[...] += fl_tot.reshape(1, 1)
    m_ref[...] += m_tot.reshape(1, 1)


def kernel(predicted_locs, predicted_scores, boxes, priors_cxcy, labels):
    B, P, C = predicted_scores.shape
    NOBJ = boxes.shape[1]

    pr = [priors_cxcy[:, k].reshape(1, P) for k in range(4)]
    lc = [predicted_locs[:, :, k].reshape(B, 1, P) for k in range(4)]
    labels3 = labels.astype(jnp.int32)[..., None]

    cls_m, npos, locsum = pl.pallas_call(
        _match_kernel,
        grid=(B,),
        in_specs=[
            pl.BlockSpec((1, NOBJ, 4), lambda i: (i, 0, 0)),
            pl.BlockSpec((1, NOBJ, 1), lambda i: (i, 0, 0)),
        ] + [pl.BlockSpec((1, P), lambda i: (0, 0)) for _ in range(4)]
          + [pl.BlockSpec((1, 1, P), lambda i: (i, 0, 0)) for _ in range(4)],
        out_specs=[
            pl.BlockSpec((1, 1, P), lambda i: (i, 0, 0)),
            pl.BlockSpec((1, 1), lambda i: (0, 0)),
            pl.BlockSpec((1, 1), lambda i: (0, 0)),
        ],
        out_shape=[
            jax.ShapeDtypeStruct((B, 1, P), jnp.int32),
            jax.ShapeDtypeStruct((1, 1), jnp.float32),
            jax.ShapeDtypeStruct((1, 1), jnp.float32),
        ],
    )(boxes, labels3, *pr, *lc)

    GRP = 32
    TPR = 313
    nblk = B * P // (TPR * GRP)
    s2 = predicted_scores.reshape(B * P // GRP, GRP, C)
    c2 = cls_m.reshape(nblk, 1, TPR * GRP)
    NS = 3
    fl_sum, m_sum = pl.pallas_call(
        _focal_kernel,
        grid=(nblk // NS,),
        in_specs=[pl.BlockSpec((TPR, GRP, C), lambda j, k=k: (NS * j + k, 0, 0))
                  for k in range(NS)]
                 + [pl.BlockSpec((1, 1, TPR * GRP),
                                 lambda j, k=k: (NS * j + k, 0, 0))
                    for k in range(NS)],
        out_specs=[
            pl.BlockSpec((1, 1), lambda j: (0, 0)),
            pl.BlockSpec((1, 1), lambda j: (0, 0)),
        ],
        out_shape=[
            jax.ShapeDtypeStruct((1, 1), jnp.float32),
            jax.ShapeDtypeStruct((1, 1), jnp.float32),
        ],
    )(s2, s2, s2, c2, c2, c2)

    conf_loss = fl_sum[0, 0] / jnp.maximum(m_sum[0, 0], 1.0)
    loc_loss = locsum[0, 0] / jnp.maximum(npos[0, 0] * 4.0, 1.0)
    return conf_loss + _REG_WEIGHT * loc_loss


# speedup vs baseline: 1.2456x; 1.0008x over previous
import jax
import jax.numpy as jnp
from jax.experimental import pallas as pl

_THRESHOLD = 0.5
_FOCAL_ALPHA = 0.25
_FOCAL_GAMMA = 2.0
_REG_WEIGHT = 1.0


def _match_kernel(boxes_ref, labels_ref, pcx_ref, pcy_ref, pw_ref, ph_ref,
                  l0_ref, l1_ref, l2_ref, l3_ref,
                  cls_ref, npos_ref, locsum_ref):
    i = pl.program_id(0)
    nobj = boxes_ref.shape[1]

    pcx = pcx_ref[...]
    pcy = pcy_ref[...]
    pw = pw_ref[...]
    ph = ph_ref[...]
    px1 = pcx - pw * 0.5
    py1 = pcy - ph * 0.5
    px2 = pcx + pw * 0.5
    py2 = pcy + ph * 0.5

    b = boxes_ref[0]
    bx1 = b[:, 0:1]
    by1 = b[:, 1:2]
    bx2 = b[:, 2:3]
    by2 = b[:, 3:4]

    wx = jnp.maximum(jnp.minimum(bx2, px2) - jnp.maximum(bx1, px1), 0.0)
    wy = jnp.maximum(jnp.minimum(by2, py2) - jnp.maximum(by1, py1), 0.0)
    inter = wx * wy
    a1 = (bx2 - bx1) * (by2 - by1)
    a2 = pw * ph
    ov = inter / (a1 + a2 - inter)

    npriors = ov.shape[1]
    iota_o = jax.lax.broadcasted_iota(jnp.int32, ov.shape, 0)
    iota_p = jax.lax.broadcasted_iota(jnp.int32, ov.shape, 1)

    maxv = jnp.max(ov, axis=0, keepdims=True)
    obj_idx = jnp.min(jnp.where(ov == maxv, iota_o, nobj),
                      axis=0, keepdims=True)
    rowmax = jnp.max(ov, axis=1, keepdims=True)
    pfo = jnp.min(jnp.where(ov == rowmax, iota_p, npriors),
                  axis=1, keepdims=True)

    match = iota_p == pfo
    chosen = jnp.max(jnp.where(match, iota_o, -1), axis=0, keepdims=True)
    forced = chosen >= 0
    obj_final = jnp.where(forced, chosen, obj_idx)
    ovl_final = jnp.where(forced, 1.0, maxv)

    selF = (iota_o == obj_final).astype(jnp.float32)
    lab = labels_ref[0]
    tbl = jnp.concatenate([bx1, by1, bx2, by2,
                           lab.astype(jnp.float32)], axis=1)
    g = jax.lax.dot_general(tbl, selF, (((0,), (0,)), ((), ())),
                            preferred_element_type=jnp.float32)
    lab_g = g[4:5, :].astype(jnp.int32)

    pos = ovl_final >= _THRESHOLD
    neg = ovl_final < _THRESHOLD - 0.1
    cls_m = jnp.where(pos, lab_g, jnp.where(neg, 0, -1))
    cls_ref[0, :, :] = cls_m

    gx1 = g[0:1, :]
    gy1 = g[1:2, :]
    gx2 = g[2:3, :]
    gy2 = g[3:4, :]
    bcx = (gx1 + gx2) * 0.5
    bcy = (gy1 + gy2) * 0.5
    bw = gx2 - gx1
    bh = gy2 - gy1
    t0 = (bcx - pcx) / pw * 10.0
    t1 = (bcy - pcy) / ph * 10.0
    t2 = jnp.log(bw / pw) * 5.0
    t3 = jnp.log(bh / ph) * 5.0

    l0 = l0_ref[0]
    l1 = l1_ref[0]
    l2 = l2_ref[0]
    l3 = l3_ref[0]
    posf = pos.astype(jnp.float32)
    ld = (jnp.abs(l0 - t0) + jnp.abs(l1 - t1)
          + jnp.abs(l2 - t2) + jnp.abs(l3 - t3)) * posf

    @pl.when(i == 0)
    def _():
        npos_ref[...] = jnp.zeros((1, 1), jnp.float32)
        locsum_ref[...] = jnp.zeros((1, 1), jnp.float32)

    npos_ref[...] += jnp.sum(posf).reshape(1, 1)
    locsum_ref[...] += jnp.sum(ld).reshape(1, 1)


def _focal_kernel(s0_ref, s1_ref, s2_ref, c0_ref, c1_ref, c2_ref,
                  fl_ref, m_ref):
    j = pl.program_id(0)
    fl_tot = jnp.zeros((), jnp.float32)
    m_tot = jnp.zeros((), jnp.float32)
    for s_ref, c_ref in ((s0_ref, c0_ref), (s1_ref, c1_ref), (s2_ref, c2_ref)):
        tpr, grp, ncls = s_ref.shape
        s = s_ref[...].reshape(tpr * grp, ncls)
        st = jnp.transpose(s)
        cls = c_ref[0]
        mx = jnp.max(st, axis=0, keepdims=True)
        e = jnp.exp(st - mx)
        se = jnp.sum(e, axis=0, keepdims=True)
        lse = mx + jnp.log(se)
        iota_c = jax.lax.broadcasted_iota(jnp.int32, st.shape, 0)
        sv = jnp.sum(jnp.where(iota_c == jnp.maximum(cls, 0), st, 0.0),
                     axis=0, keepdims=True)
        lpt = sv - lse
        pt = jnp.exp(lpt)
        mm = (cls >= 0).astype(jnp.float32)
        alpha = jnp.where(cls > 0, _FOCAL_ALPHA, 1.0 - _FOCAL_ALPHA)
        om = 1.0 - pt
        fl_tot += jnp.sum(-alpha * om * om * lpt * mm)
        m_tot += jnp.sum(mm)

    @pl.when(j == 0)
    def _():
        fl_ref[...] = jnp.zeros((1, 1), jnp.float32)
        m_ref[...] = jnp.zeros((1, 1), jnp.float32)

    fl_ref[...] += fl_tot.reshape(1, 1)
    m_ref[...] += m_tot.reshape(1, 1)


def kernel(predicted_locs, predicted_scores, boxes, priors_cxcy, labels):
    B, P, C = predicted_scores.shape
    NOBJ = boxes.shape[1]

    pr = [priors_cxcy[:, k].reshape(1, P) for k in range(4)]
    lc = [predicted_locs[:, :, k].reshape(B, 1, P) for k in range(4)]
    labels3 = labels.astype(jnp.int32)[..., None]

    cls_m, npos, locsum = pl.pallas_call(
        _match_kernel,
        grid=(B,),
        in_specs=[
            pl.BlockSpec((1, NOBJ, 4), lambda i: (i, 0, 0)),
            pl.BlockSpec((1, NOBJ, 1), lambda i: (i, 0, 0)),
        ] + [pl.BlockSpec((1, P), lambda i: (0, 0)) for _ in range(4)]
          + [pl.BlockSpec((1, 1, P), lambda i: (i, 0, 0)) for _ in range(4)],
        out_specs=[
            pl.BlockSpec((1, 1, P), lambda i: (i, 0, 0)),
            pl.BlockSpec((1, 1), lambda i: (0, 0)),
            pl.BlockSpec((1, 1), lambda i: (0, 0)),
        ],
        out_shape=[
            jax.ShapeDtypeStruct((B, 1, P), jnp.int32),
            jax.ShapeDtypeStruct((1, 1), jnp.float32),
            jax.ShapeDtypeStruct((1, 1), jnp.float32),
        ],
    )(boxes, labels3, *pr, *lc)

    GRP = 32
    TPR = 313
    nblk = B * P // (TPR * GRP)
    s2 = predicted_scores.reshape(B * P // GRP, GRP, C)
    c2 = cls_m.reshape(nblk, 1, TPR * GRP)
    NS = 3
    fl_sum, m_sum = pl.pallas_call(
        _focal_kernel,
        grid=(nblk // NS,),
        in_specs=[pl.BlockSpec((TPR, GRP, C), lambda j, k=k: (NS * j + k, 0, 0))
                  for k in range(NS)]
                 + [pl.BlockSpec((1, 1, TPR * GRP),
                                 lambda j, k=k: (NS * j + k, 0, 0))
                    for k in range(NS)],
        out_specs=[
            pl.BlockSpec((1, 1), lambda j: (0, 0)),
            pl.BlockSpec((1, 1), lambda j: (0, 0)),
        ],
        out_shape=[
            jax.ShapeDtypeStruct((1, 1), jnp.float32),
            jax.ShapeDtypeStruct((1, 1), jnp.float32),
        ],
    )(s2, s2, s2, c2, c2, c2)

    conf_loss = fl_sum[0, 0] / jnp.maximum(m_sum[0, 0], 1.0)
    loc_loss = locsum[0, 0] / jnp.maximum(npos[0, 0] * 4.0, 1.0)
    return conf_loss + _REG_WEIGHT * loc_loss
